# initial kernel scaffold (unmeasured)
import functools

import jax
import jax.numpy as jnp
from jax import lax
from jax.experimental import pallas as pl
from jax.experimental.pallas import tpu as pltpu

N_Z = 4
B, S, H, Dh, Dr = 4, 256, 32, 128, 64
D = 4096
DC_SH = 128


def _mm_body(a_ref, b_ref, o_ref):
    @pl.when(pl.program_id(2) == 0)
    def _():
        o_ref[...] = jnp.zeros_like(o_ref)

    o_ref[...] += jnp.dot(
        a_ref[...], b_ref[...], preferred_element_type=jnp.float32
    )


def mm(a, b, bm=512, bn=512, bk=512):
    M, K = a.shape
    K2, N = b.shape
    assert K == K2
    bm, bn, bk = min(bm, M), min(bn, N), min(bk, K)
    grid = (M // bm, N // bn, K // bk)
    return pl.pallas_call(
        _mm_body,
        grid=grid,
        in_specs=[
            pl.BlockSpec((bm, bk), lambda i, j, k: (i, k)),
            pl.BlockSpec((bk, bn), lambda i, j, k: (k, j)),
        ],
        out_specs=pl.BlockSpec((bm, bn), lambda i, j, k: (i, j)),
        out_shape=jax.ShapeDtypeStruct((M, N), jnp.float32),
        compiler_params=pltpu.CompilerParams(
            dimension_semantics=("parallel", "parallel", "arbitrary")
        ),
    )(a, b)


def _gather_body(c_ref, uk_ref, uv_ref, oc_ref, ouk_ref, ouv_ref,
                 send_sems, recv_sems):
    my_x = lax.axis_index("x")
    my_y = lax.axis_index("y")
    my_z = lax.axis_index("z")
    left = (my_z - 1) % N_Z
    right = (my_z + 1) % N_Z

    barrier_sem = pltpu.get_barrier_semaphore()
    for nbr in (left, right):
        pl.semaphore_signal(
            barrier_sem, inc=1,
            device_id=(my_x, my_y, nbr),
            device_id_type=pl.DeviceIdType.MESH,
        )
    pl.semaphore_wait(barrier_sem, 2)

    oc_ref[0] = c_ref[...]
    ouk_ref[0] = uk_ref[...]
    ouv_ref[0] = uv_ref[...]

    for h in range(N_Z - 1):
        rdmas = []
        for t, ref in enumerate((oc_ref, ouk_ref, ouv_ref)):
            r = pltpu.make_async_remote_copy(
                src_ref=ref.at[h],
                dst_ref=ref.at[h + 1],
                send_sem=send_sems.at[h, t],
                recv_sem=recv_sems.at[h, t],
                device_id=(my_x, my_y, right),
                device_id_type=pl.DeviceIdType.MESH,
            )
            r.start()
            rdmas.append(r)
        for r in rdmas:
            r.wait()


def gather3(c, uk, uv):
    n_sem = (N_Z - 1, 3)
    return pl.pallas_call(
        _gather_body,
        out_shape=[
            jax.ShapeDtypeStruct((N_Z, B * S, DC_SH), jnp.float32),
            jax.ShapeDtypeStruct((N_Z, DC_SH, D), jnp.float32),
            jax.ShapeDtypeStruct((N_Z, DC_SH, D), jnp.float32),
        ],
        in_specs=[pl.BlockSpec(memory_space=pltpu.VMEM)] * 3,
        out_specs=[pl.BlockSpec(memory_space=pltpu.VMEM)] * 3,
        scratch_shapes=[
            pltpu.SemaphoreType.DMA(n_sem),
            pltpu.SemaphoreType.DMA(n_sem),
        ],
        compiler_params=pltpu.CompilerParams(collective_id=0),
    )(c, uk, uv)


_SCALE = (Dh + Dr) ** -0.5


def _attn_body(q_ref, k_ref, v_ref, qr_ref, kr_ref, o_ref):
    s = lax.dot_general(
        q_ref[...], k_ref[...], (((1,), (1,)), ((), ())),
        preferred_element_type=jnp.float32,
    )
    s += lax.dot_general(
        qr_ref[...], kr_ref[...], (((1,), (1,)), ((), ())),
        preferred_element_type=jnp.float32,
    )
    s *= _SCALE
    m = jnp.max(s, axis=1, keepdims=True)
    p = jnp.exp(s - m)
    p = p / jnp.sum(p, axis=1, keepdims=True)
    o_ref[...] = jnp.dot(p, v_ref[...], preferred_element_type=jnp.float32)


def attention(Q, K, V, Qr, Kr):
    return pl.pallas_call(
        _attn_body,
        grid=(B, H),
        in_specs=[
            pl.BlockSpec((S, Dh), lambda b, h: (b, h)),
            pl.BlockSpec((S, Dh), lambda b, h: (b, h)),
            pl.BlockSpec((S, Dh), lambda b, h: (b, h)),
            pl.BlockSpec((S, Dr), lambda b, h: (b, h)),
            pl.BlockSpec((S, Dr), lambda b, h: (b, 0)),
        ],
        out_specs=pl.BlockSpec((S, Dh), lambda b, h: (b, h)),
        out_shape=jax.ShapeDtypeStruct((B * S, H * Dh), jnp.float32),
        compiler_params=pltpu.CompilerParams(
            dimension_semantics=("parallel", "parallel")
        ),
    )(Q, K, V, Qr, Kr)


def kernel(x, Wdkv, Wuk, Wuv, Wq, Wqr, Wkr, Wo):
    x2 = x.reshape(B * S, D)

    c = mm(x2, Wdkv)

    c_all, uk_all, uv_all = gather3(c, Wuk, Wuv)

    c_full = c_all.transpose(1, 0, 2).reshape(B * S, N_Z * DC_SH)
    uk_full = uk_all.reshape(N_Z * DC_SH, D)
    uv_full = uv_all.reshape(N_Z * DC_SH, D)

    K = mm(c_full, uk_full)
    V = mm(c_full, uv_full)
    Q = mm(x2, Wq)
    Qr = mm(x2, Wqr)
    Kr = mm(x2, Wkr)

    O = attention(Q, K, V, Qr, Kr)
    out = mm(O, Wo)
    return out.reshape(B, S, D)


# baseline (device time: 688634 ns/iter reference)
import functools

import jax
import jax.numpy as jnp
from jax import lax
from jax.experimental import pallas as pl
from jax.experimental.pallas import tpu as pltpu

N_Z = 4
B, S, H, Dh, Dr = 4, 256, 32, 128, 64
D = 4096
DC_SH = 128


def _mm_body(a_ref, b_ref, o_ref):
    @pl.when(pl.program_id(2) == 0)
    def _():
        o_ref[...] = jnp.zeros_like(o_ref)

    o_ref[...] += jnp.dot(
        a_ref[...], b_ref[...], preferred_element_type=jnp.float32
    )


def mm(a, b, bm=512, bn=512, bk=512):
    M, K = a.shape
    K2, N = b.shape
    assert K == K2
    bm, bn, bk = min(bm, M), min(bn, N), min(bk, K)
    grid = (M // bm, N // bn, K // bk)
    return pl.pallas_call(
        _mm_body,
        grid=grid,
        in_specs=[
            pl.BlockSpec((bm, bk), lambda i, j, k: (i, k)),
            pl.BlockSpec((bk, bn), lambda i, j, k: (k, j)),
        ],
        out_specs=pl.BlockSpec((bm, bn), lambda i, j, k: (i, j)),
        out_shape=jax.ShapeDtypeStruct((M, N), jnp.float32),
        compiler_params=pltpu.CompilerParams(
            dimension_semantics=("parallel", "parallel", "arbitrary")
        ),
    )(a, b)


def _gather_body(c_ref, uk_ref, uv_ref, oc_ref, ouk_ref, ouv_ref,
                 send_sems, recv_sems):
    my_x = lax.axis_index("x")
    my_y = lax.axis_index("y")
    my_z = lax.axis_index("z")
    left = (my_z - 1) % N_Z
    right = (my_z + 1) % N_Z

    barrier_sem = pltpu.get_barrier_semaphore()
    for nbr in (left, right):
        pl.semaphore_signal(
            barrier_sem, inc=1,
            device_id=(my_x, my_y, nbr),
            device_id_type=pl.DeviceIdType.MESH,
        )
    pl.semaphore_wait(barrier_sem, 2)

    oc_ref[0] = c_ref[...]
    ouk_ref[0] = uk_ref[...]
    ouv_ref[0] = uv_ref[...]

    for h in range(N_Z - 1):
        rdmas = []
        for t, ref in enumerate((oc_ref, ouk_ref, ouv_ref)):
            r = pltpu.make_async_remote_copy(
                src_ref=ref.at[h],
                dst_ref=ref.at[h + 1],
                send_sem=send_sems.at[h, t],
                recv_sem=recv_sems.at[h, t],
                device_id=(my_x, my_y, right),
                device_id_type=pl.DeviceIdType.MESH,
            )
            r.start()
            rdmas.append(r)
        for r in rdmas:
            r.wait()


def gather3(c, uk, uv):
    n_sem = (N_Z - 1, 3)
    return pl.pallas_call(
        _gather_body,
        out_shape=[
            jax.ShapeDtypeStruct((N_Z, B * S, DC_SH), jnp.float32),
            jax.ShapeDtypeStruct((N_Z, DC_SH, D), jnp.float32),
            jax.ShapeDtypeStruct((N_Z, DC_SH, D), jnp.float32),
        ],
        in_specs=[pl.BlockSpec(memory_space=pltpu.VMEM)] * 3,
        out_specs=[pl.BlockSpec(memory_space=pltpu.VMEM)] * 3,
        scratch_shapes=[
            pltpu.SemaphoreType.DMA(n_sem),
            pltpu.SemaphoreType.DMA(n_sem),
        ],
        compiler_params=pltpu.CompilerParams(collective_id=0),
    )(c, uk, uv)


_SCALE = (Dh + Dr) ** -0.5


def _attn_body(q_ref, k_ref, v_ref, qr_ref, kr_ref, o_ref):
    s = lax.dot_general(
        q_ref[...], k_ref[...], (((1,), (1,)), ((), ())),
        preferred_element_type=jnp.float32,
    )
    s += lax.dot_general(
        qr_ref[...], kr_ref[...], (((1,), (1,)), ((), ())),
        preferred_element_type=jnp.float32,
    )
    s *= _SCALE
    m = jnp.max(s, axis=1, keepdims=True)
    p = jnp.exp(s - m)
    p = p / jnp.sum(p, axis=1, keepdims=True)
    o_ref[...] = jnp.dot(p, v_ref[...], preferred_element_type=jnp.float32)


def attention(Q, K, V, Qr_t, Kr):
    return pl.pallas_call(
        _attn_body,
        grid=(B, H),
        in_specs=[
            pl.BlockSpec((S, Dh), lambda b, h: (b, h)),
            pl.BlockSpec((S, Dh), lambda b, h: (b, h)),
            pl.BlockSpec((S, Dh), lambda b, h: (b, h)),
            pl.BlockSpec((S, Dr), lambda b, h: (h * B + b, 0)),
            pl.BlockSpec((S, Dr), lambda b, h: (b, 0)),
        ],
        out_specs=pl.BlockSpec((S, Dh), lambda b, h: (b, h)),
        out_shape=jax.ShapeDtypeStruct((B * S, H * Dh), jnp.float32),
        compiler_params=pltpu.CompilerParams(
            dimension_semantics=("parallel", "parallel")
        ),
    )(Q, K, V, Qr_t, Kr)


def kernel(x, Wdkv, Wuk, Wuv, Wq, Wqr, Wkr, Wo):
    x2 = x.reshape(B * S, D)

    c = mm(x2, Wdkv)

    c_all, uk_all, uv_all = gather3(c, Wuk, Wuv)

    c_full = c_all.transpose(1, 0, 2).reshape(B * S, N_Z * DC_SH)
    uk_full = uk_all.reshape(N_Z * DC_SH, D)
    uv_full = uv_all.reshape(N_Z * DC_SH, D)

    K = mm(c_full, uk_full)
    V = mm(c_full, uv_full)
    Q = mm(x2, Wq)
    Qr = mm(x2, Wqr)
    Kr = mm(x2, Wkr)

    Qr_t = Qr.reshape(B * S, H, Dr).transpose(1, 0, 2).reshape(H * B * S, Dr)
    O = attention(Q, K, V, Qr_t, Kr)
    out = mm(O, Wo)
    return out.reshape(B, S, D)


# device time: 373462 ns/iter; 1.8439x vs baseline; 1.8439x over previous
import jax
import jax.numpy as jnp
from jax import lax
from jax.experimental import pallas as pl
from jax.experimental.pallas import tpu as pltpu

N_Z = 4
B, S, H, Dh, Dr = 4, 256, 32, 128, 64
D = 4096
DC_SH = 128
H_L = H // N_Z
WQ_L = H_L * Dh
WR_L = H_L * Dr


def _mm_body(a_ref, b_ref, o_ref):
    @pl.when(pl.program_id(2) == 0)
    def _():
        o_ref[...] = jnp.zeros_like(o_ref)

    o_ref[...] += jnp.dot(
        a_ref[...], b_ref[...], preferred_element_type=jnp.float32
    )


def mm(a, b, bm=512, bn=512, bk=512):
    M, K = a.shape
    K2, N = b.shape
    assert K == K2
    bm, bn, bk = min(bm, M), min(bn, N), min(bk, K)
    grid = (M // bm, N // bn, K // bk)
    return pl.pallas_call(
        _mm_body,
        grid=grid,
        in_specs=[
            pl.BlockSpec((bm, bk), lambda i, j, k: (i, k)),
            pl.BlockSpec((bk, bn), lambda i, j, k: (k, j)),
        ],
        out_specs=pl.BlockSpec((bm, bn), lambda i, j, k: (i, j)),
        out_shape=jax.ShapeDtypeStruct((M, N), jnp.float32),
        compiler_params=pltpu.CompilerParams(
            dimension_semantics=("parallel", "parallel", "arbitrary")
        ),
    )(a, b)


def _exch_body(c_ref, uko_ref, uvo_ref,
               uk1_ref, uk2_ref, uk3_ref, uv1_ref, uv2_ref, uv3_ref,
               oc_ref, ouk_ref, ouv_ref, send_sems, recv_sems):
    my_x = lax.axis_index("x")
    my_y = lax.axis_index("y")
    my_z = lax.axis_index("z")

    barrier_sem = pltpu.get_barrier_semaphore()
    for d in range(1, N_Z):
        pl.semaphore_signal(
            barrier_sem, inc=1,
            device_id=(my_x, my_y, (my_z + d) % N_Z),
            device_id_type=pl.DeviceIdType.MESH,
        )
    pl.semaphore_wait(barrier_sem, N_Z - 1)

    oc_ref[0] = c_ref[...]
    ouk_ref[0] = uko_ref[...]
    ouv_ref[0] = uvo_ref[...]

    uks = (uk1_ref, uk2_ref, uk3_ref)
    uvs = (uv1_ref, uv2_ref, uv3_ref)
    rdmas = []
    for d in range(1, N_Z):
        peer = (my_z + d) % N_Z
        for t, (src, dst) in enumerate((
            (c_ref, oc_ref),
            (uks[d - 1], ouk_ref),
            (uvs[d - 1], ouv_ref),
        )):
            r = pltpu.make_async_remote_copy(
                src_ref=src,
                dst_ref=dst.at[d],
                send_sem=send_sems.at[d - 1, t],
                recv_sem=recv_sems.at[d - 1, t],
                device_id=(my_x, my_y, peer),
                device_id_type=pl.DeviceIdType.MESH,
            )
            r.start()
            rdmas.append(r)
    for r in rdmas:
        r.wait()


def exchange(c, uk_own, uv_own, uk_s, uv_s):
    n_sem = (N_Z - 1, 3)
    return pl.pallas_call(
        _exch_body,
        out_shape=[
            jax.ShapeDtypeStruct((N_Z, B * S, DC_SH), jnp.float32),
            jax.ShapeDtypeStruct((N_Z, DC_SH, WQ_L), jnp.float32),
            jax.ShapeDtypeStruct((N_Z, DC_SH, WQ_L), jnp.float32),
        ],
        in_specs=[pl.BlockSpec(memory_space=pltpu.VMEM)] * 9,
        out_specs=[pl.BlockSpec(memory_space=pltpu.VMEM)] * 3,
        scratch_shapes=[
            pltpu.SemaphoreType.DMA(n_sem),
            pltpu.SemaphoreType.DMA(n_sem),
        ],
        compiler_params=pltpu.CompilerParams(collective_id=0),
    )(c, uk_own, uv_own, *uk_s, *uv_s)


_SCALE = (Dh + Dr) ** -0.5


def _attn_body(q_ref, k_ref, v_ref, qr_ref, kr_ref, o_ref):
    s = lax.dot_general(
        q_ref[...], k_ref[...], (((1,), (1,)), ((), ())),
        preferred_element_type=jnp.float32,
    )
    s += lax.dot_general(
        qr_ref[...], kr_ref[...], (((1,), (1,)), ((), ())),
        preferred_element_type=jnp.float32,
    )
    s *= _SCALE
    m = jnp.max(s, axis=1, keepdims=True)
    p = jnp.exp(s - m)
    p = p / jnp.sum(p, axis=1, keepdims=True)
    o_ref[...] = jnp.dot(p, v_ref[...], preferred_element_type=jnp.float32)


def attention(Q, K, V, Qr_t, Kr):
    return pl.pallas_call(
        _attn_body,
        grid=(B, H_L),
        in_specs=[
            pl.BlockSpec((S, Dh), lambda b, h: (b, h)),
            pl.BlockSpec((S, Dh), lambda b, h: (b, h)),
            pl.BlockSpec((S, Dh), lambda b, h: (b, h)),
            pl.BlockSpec((S, Dr), lambda b, h: (h * B + b, 0)),
            pl.BlockSpec((S, Dr), lambda b, h: (b, 0)),
        ],
        out_specs=pl.BlockSpec((S, Dh), lambda b, h: (b, h)),
        out_shape=jax.ShapeDtypeStruct((B * S, H_L * Dh), jnp.float32),
        compiler_params=pltpu.CompilerParams(
            dimension_semantics=("parallel", "parallel")
        ),
    )(Q, K, V, Qr_t, Kr)


def _agwo_body(o_ref, wo_ref, out_ref, slots, wo_buf,
               send_sems, recv_sems, wo_sem):
    my_x = lax.axis_index("x")
    my_y = lax.axis_index("y")
    my_z = lax.axis_index("z")
    left = (my_z - 1) % N_Z
    right = (my_z + 1) % N_Z

    barrier_sem = pltpu.get_barrier_semaphore()
    for nbr in (left, right):
        pl.semaphore_signal(
            barrier_sem, inc=1,
            device_id=(my_x, my_y, nbr),
            device_id_type=pl.DeviceIdType.MESH,
        )
    pl.semaphore_wait(barrier_sem, 2)

    slots[0] = o_ref[...]

    def wo_dma(h):
        origin = (my_z - h) % N_Z
        return pltpu.make_async_copy(
            wo_ref.at[pl.ds(origin * WQ_L, WQ_L), :], wo_buf, wo_sem
        )

    wo_dma(0).start()
    for h in range(N_Z):
        if h < N_Z - 1:
            rdma = pltpu.make_async_remote_copy(
                src_ref=slots.at[h],
                dst_ref=slots.at[h + 1],
                send_sem=send_sems.at[h],
                recv_sem=recv_sems.at[h],
                device_id=(my_x, my_y, right),
                device_id_type=pl.DeviceIdType.MESH,
            )
            rdma.start()
        wo_dma(h).wait()
        prod = jnp.dot(
            slots[h], wo_buf[...], preferred_element_type=jnp.float32
        )
        if h == 0:
            out_ref[...] = prod
        else:
            out_ref[...] += prod
        if h < N_Z - 1:
            rdma.wait()
            wo_dma(h + 1).start()


def ag_wo(O_my, Wo):
    return pl.pallas_call(
        _agwo_body,
        out_shape=jax.ShapeDtypeStruct((B * S, D), jnp.float32),
        in_specs=[
            pl.BlockSpec(memory_space=pltpu.VMEM),
            pl.BlockSpec(memory_space=pltpu.MemorySpace.HBM),
        ],
        out_specs=pl.BlockSpec(memory_space=pltpu.VMEM),
        scratch_shapes=[
            pltpu.VMEM((N_Z, B * S, WQ_L), jnp.float32),
            pltpu.VMEM((WQ_L, D), jnp.float32),
            pltpu.SemaphoreType.DMA((N_Z - 1,)),
            pltpu.SemaphoreType.DMA((N_Z - 1,)),
            pltpu.SemaphoreType.DMA,
        ],
        compiler_params=pltpu.CompilerParams(collective_id=1),
    )(O_my, Wo)


def kernel(x, Wdkv, Wuk, Wuv, Wq, Wqr, Wkr, Wo):
    my_z = lax.axis_index("z")
    x2 = x.reshape(B * S, D)

    def cols(W, pos, w):
        return lax.dynamic_slice(W, (0, pos * w), (W.shape[0], w))

    c = mm(x2, Wdkv)

    uk_own = cols(Wuk, my_z, WQ_L)
    uv_own = cols(Wuv, my_z, WQ_L)
    uk_s = [cols(Wuk, (my_z + d) % N_Z, WQ_L) for d in range(1, N_Z)]
    uv_s = [cols(Wuv, (my_z + d) % N_Z, WQ_L) for d in range(1, N_Z)]
    c_all, uk_all, uv_all = exchange(c, uk_own, uv_own, uk_s, uv_s)

    c_full = c_all.transpose(1, 0, 2).reshape(B * S, N_Z * DC_SH)
    uk_my = uk_all.reshape(N_Z * DC_SH, WQ_L)
    uv_my = uv_all.reshape(N_Z * DC_SH, WQ_L)

    K = mm(c_full, uk_my)
    V = mm(c_full, uv_my)
    Q = mm(x2, cols(Wq, my_z, WQ_L))
    Qr = mm(x2, cols(Wqr, my_z, WR_L))
    Kr = mm(x2, Wkr)

    Qr_t = Qr.reshape(B * S, H_L, Dr).transpose(1, 0, 2).reshape(
        H_L * B * S, Dr
    )
    O = attention(Q, K, V, Qr_t, Kr)

    out = ag_wo(O, Wo)
    return out.reshape(B, S, D)


# device time: 261207 ns/iter; 2.6364x vs baseline; 1.4298x over previous
import jax
import jax.numpy as jnp
from jax import lax
from jax.experimental import pallas as pl
from jax.experimental.pallas import tpu as pltpu

N_Z = 4
B, S, H, Dh, Dr = 4, 256, 32, 128, 64
D = 4096
DC_SH = 128
H_L = H // N_Z
WQ_L = H_L * Dh
WR_L = H_L * Dr


def _mm_body(a_ref, b_ref, o_ref):
    @pl.when(pl.program_id(2) == 0)
    def _():
        o_ref[...] = jnp.zeros_like(o_ref)

    o_ref[...] += jnp.dot(
        a_ref[...], b_ref[...], preferred_element_type=jnp.float32
    )


def mm(a, b, bm=512, bn=512, bk=512):
    M, K = a.shape
    K2, N = b.shape
    assert K == K2
    bm, bn, bk = min(bm, M), min(bn, N), min(bk, K)
    grid = (M // bm, N // bn, K // bk)
    return pl.pallas_call(
        _mm_body,
        grid=grid,
        in_specs=[
            pl.BlockSpec((bm, bk), lambda i, j, k: (i, k)),
            pl.BlockSpec((bk, bn), lambda i, j, k: (k, j)),
        ],
        out_specs=pl.BlockSpec((bm, bn), lambda i, j, k: (i, j)),
        out_shape=jax.ShapeDtypeStruct((M, N), jnp.float32),
        compiler_params=pltpu.CompilerParams(
            dimension_semantics=("parallel", "parallel", "arbitrary")
        ),
    )(a, b)


def _exch_body(c_ref, uko_ref, uvo_ref,
               uk1_ref, uk2_ref, uk3_ref, uv1_ref, uv2_ref, uv3_ref,
               oc_ref, ouk_ref, ouv_ref, send_sems, recv_sems):
    my_x = lax.axis_index("x")
    my_y = lax.axis_index("y")
    my_z = lax.axis_index("z")

    barrier_sem = pltpu.get_barrier_semaphore()
    for d in range(1, N_Z):
        pl.semaphore_signal(
            barrier_sem, inc=1,
            device_id=(my_x, my_y, (my_z + d) % N_Z),
            device_id_type=pl.DeviceIdType.MESH,
        )
    pl.semaphore_wait(barrier_sem, N_Z - 1)

    oc_ref[0] = c_ref[...]
    ouk_ref[0] = uko_ref[...]
    ouv_ref[0] = uvo_ref[...]

    uks = (uk1_ref, uk2_ref, uk3_ref)
    uvs = (uv1_ref, uv2_ref, uv3_ref)
    rdmas = []
    for d in range(1, N_Z):
        peer = (my_z + d) % N_Z
        for t, (src, dst) in enumerate((
            (c_ref, oc_ref),
            (uks[d - 1], ouk_ref),
            (uvs[d - 1], ouv_ref),
        )):
            r = pltpu.make_async_remote_copy(
                src_ref=src,
                dst_ref=dst.at[d],
                send_sem=send_sems.at[d - 1, t],
                recv_sem=recv_sems.at[d - 1, t],
                device_id=(my_x, my_y, peer),
                device_id_type=pl.DeviceIdType.MESH,
            )
            r.start()
            rdmas.append(r)
    for r in rdmas:
        r.wait()


def exchange(c, uk_own, uv_own, uk_s, uv_s):
    n_sem = (N_Z - 1, 3)
    return pl.pallas_call(
        _exch_body,
        out_shape=[
            jax.ShapeDtypeStruct((N_Z, B * S, DC_SH), jnp.bfloat16),
            jax.ShapeDtypeStruct((N_Z, DC_SH, WQ_L), jnp.bfloat16),
            jax.ShapeDtypeStruct((N_Z, DC_SH, WQ_L), jnp.bfloat16),
        ],
        in_specs=[pl.BlockSpec(memory_space=pltpu.VMEM)] * 9,
        out_specs=[pl.BlockSpec(memory_space=pltpu.VMEM)] * 3,
        scratch_shapes=[
            pltpu.SemaphoreType.DMA(n_sem),
            pltpu.SemaphoreType.DMA(n_sem),
        ],
        compiler_params=pltpu.CompilerParams(collective_id=0),
    )(c, uk_own, uv_own, *uk_s, *uv_s)


_SCALE = (Dh + Dr) ** -0.5


def _attn_body(q_ref, k_ref, v_ref, qr_ref, kr_ref, o_ref):
    s = lax.dot_general(
        q_ref[...], k_ref[...], (((1,), (1,)), ((), ())),
        preferred_element_type=jnp.float32,
    )
    s += lax.dot_general(
        qr_ref[...], kr_ref[...], (((1,), (1,)), ((), ())),
        preferred_element_type=jnp.float32,
    )
    s *= _SCALE
    m = jnp.max(s, axis=1, keepdims=True)
    p = jnp.exp(s - m)
    p = p / jnp.sum(p, axis=1, keepdims=True)
    o_ref[...] = jnp.dot(
        p.astype(jnp.bfloat16), v_ref[...],
        preferred_element_type=jnp.float32,
    ).astype(jnp.bfloat16)


def attention(Q, K, V, Qr_t, Kr):
    return pl.pallas_call(
        _attn_body,
        grid=(B, H_L),
        in_specs=[
            pl.BlockSpec((S, Dh), lambda b, h: (b, h)),
            pl.BlockSpec((S, Dh), lambda b, h: (b, h)),
            pl.BlockSpec((S, Dh), lambda b, h: (b, h)),
            pl.BlockSpec((S, Dr), lambda b, h: (h * B + b, 0)),
            pl.BlockSpec((S, Dr), lambda b, h: (b, 0)),
        ],
        out_specs=pl.BlockSpec((S, Dh), lambda b, h: (b, h)),
        out_shape=jax.ShapeDtypeStruct((B * S, H_L * Dh), jnp.bfloat16),
        compiler_params=pltpu.CompilerParams(
            dimension_semantics=("parallel", "parallel")
        ),
    )(Q, K, V, Qr_t, Kr)


def _agwo_body(o_ref, wo_ref, out_ref, slots, wo_buf,
               send_sems, recv_sems, wo_sem):
    my_x = lax.axis_index("x")
    my_y = lax.axis_index("y")
    my_z = lax.axis_index("z")
    left = (my_z - 1) % N_Z
    right = (my_z + 1) % N_Z

    barrier_sem = pltpu.get_barrier_semaphore()
    for nbr in (left, right):
        pl.semaphore_signal(
            barrier_sem, inc=1,
            device_id=(my_x, my_y, nbr),
            device_id_type=pl.DeviceIdType.MESH,
        )
    pl.semaphore_wait(barrier_sem, 2)

    slots[0] = o_ref[...]

    def wo_dma(h):
        origin = (my_z - h) % N_Z
        return pltpu.make_async_copy(
            wo_ref.at[pl.ds(origin * WQ_L, WQ_L), :], wo_buf, wo_sem
        )

    wo_dma(0).start()
    for h in range(N_Z):
        if h < N_Z - 1:
            rdma = pltpu.make_async_remote_copy(
                src_ref=slots.at[h],
                dst_ref=slots.at[h + 1],
                send_sem=send_sems.at[h],
                recv_sem=recv_sems.at[h],
                device_id=(my_x, my_y, right),
                device_id_type=pl.DeviceIdType.MESH,
            )
            rdma.start()
        wo_dma(h).wait()
        prod = jnp.dot(
            slots[h], wo_buf[...], preferred_element_type=jnp.float32
        )
        if h == 0:
            out_ref[...] = prod
        else:
            out_ref[...] += prod
        if h < N_Z - 1:
            rdma.wait()
            wo_dma(h + 1).start()


def ag_wo(O_my, Wo):
    return pl.pallas_call(
        _agwo_body,
        out_shape=jax.ShapeDtypeStruct((B * S, D), jnp.float32),
        in_specs=[
            pl.BlockSpec(memory_space=pltpu.VMEM),
            pl.BlockSpec(memory_space=pltpu.MemorySpace.HBM),
        ],
        out_specs=pl.BlockSpec(memory_space=pltpu.VMEM),
        scratch_shapes=[
            pltpu.VMEM((N_Z, B * S, WQ_L), jnp.bfloat16),
            pltpu.VMEM((WQ_L, D), jnp.bfloat16),
            pltpu.SemaphoreType.DMA((N_Z - 1,)),
            pltpu.SemaphoreType.DMA((N_Z - 1,)),
            pltpu.SemaphoreType.DMA,
        ],
        compiler_params=pltpu.CompilerParams(collective_id=1),
    )(O_my, Wo)


def kernel(x, Wdkv, Wuk, Wuv, Wq, Wqr, Wkr, Wo):
    my_z = lax.axis_index("z")
    bf = jnp.bfloat16
    x2 = x.reshape(B * S, D).astype(bf)

    def cols(W, pos, w):
        return lax.dynamic_slice(W, (0, pos * w), (W.shape[0], w)).astype(bf)

    c = mm(x2, Wdkv.astype(bf))

    uk_own = cols(Wuk, my_z, WQ_L)
    uv_own = cols(Wuv, my_z, WQ_L)
    uk_s = [cols(Wuk, (my_z + d) % N_Z, WQ_L) for d in range(1, N_Z)]
    uv_s = [cols(Wuv, (my_z + d) % N_Z, WQ_L) for d in range(1, N_Z)]
    c_all, uk_all, uv_all = exchange(
        c.astype(bf), uk_own, uv_own, uk_s, uv_s
    )

    c_full = c_all.transpose(1, 0, 2).reshape(B * S, N_Z * DC_SH)
    uk_my = uk_all.reshape(N_Z * DC_SH, WQ_L)
    uv_my = uv_all.reshape(N_Z * DC_SH, WQ_L)

    K = mm(c_full, uk_my)
    V = mm(c_full, uv_my)
    Q = mm(x2, cols(Wq, my_z, WQ_L))
    Qr = mm(x2, cols(Wqr, my_z, WR_L))
    Kr = mm(x2, Wkr.astype(bf))

    Qr_t = Qr.reshape(B * S, H_L, Dr).transpose(1, 0, 2).reshape(
        H_L * B * S, Dr
    )
    O = attention(Q.astype(bf), K.astype(bf), V.astype(bf),
                  Qr_t.astype(bf), Kr.astype(bf))

    out = ag_wo(O, Wo.astype(bf))
    return out.reshape(B, S, D)


# device time: 241311 ns/iter; 2.8537x vs baseline; 1.0824x over previous
import jax
import jax.numpy as jnp
from jax import lax
from jax.experimental import pallas as pl
from jax.experimental.pallas import tpu as pltpu

N_Z = 4
B, S, H, Dh, Dr = 4, 256, 32, 128, 64
D = 4096
DC_SH = 128
H_L = H // N_Z
WQ_L = H_L * Dh
WR_L = H_L * Dr


def _mm_body(off_ref, a_ref, b_ref, o_ref):
    @pl.when(pl.program_id(2) == 0)
    def _():
        o_ref[...] = jnp.zeros_like(o_ref)

    a = a_ref[...]
    b = b_ref[...]
    if a.dtype != jnp.bfloat16:
        a = a.astype(jnp.bfloat16)
    if b.dtype != jnp.bfloat16:
        b = b.astype(jnp.bfloat16)
    o_ref[...] += jnp.dot(a, b, preferred_element_type=jnp.float32)


def mm(a, b, col_off=None, n=None, bm=512, bn=512, bk=512):
    M, K = a.shape
    K2, N = b.shape
    assert K == K2
    n = N if n is None else n
    bm, bn, bk = min(bm, M), min(bn, n), min(bk, K)
    if col_off is None:
        col_off = 0
    off_blk = jnp.reshape(col_off // bn, (1,)).astype(jnp.int32)
    grid = (M // bm, n // bn, K // bk)
    return pl.pallas_call(
        _mm_body,
        grid_spec=pltpu.PrefetchScalarGridSpec(
            num_scalar_prefetch=1,
            grid=grid,
            in_specs=[
                pl.BlockSpec((bm, bk), lambda i, j, k, off: (i, k)),
                pl.BlockSpec((bk, bn), lambda i, j, k, off: (k, off[0] + j)),
            ],
            out_specs=pl.BlockSpec((bm, bn), lambda i, j, k, off: (i, j)),
        ),
        out_shape=jax.ShapeDtypeStruct((M, n), jnp.float32),
        compiler_params=pltpu.CompilerParams(
            dimension_semantics=("parallel", "parallel", "arbitrary")
        ),
    )(off_blk, a, b)


def _exch_body(c_ref, uko_ref, uvo_ref,
               uk1_ref, uk2_ref, uk3_ref, uv1_ref, uv2_ref, uv3_ref,
               oc_ref, ouk_ref, ouv_ref, send_sems, recv_sems):
    my_x = lax.axis_index("x")
    my_y = lax.axis_index("y")
    my_z = lax.axis_index("z")

    barrier_sem = pltpu.get_barrier_semaphore()
    for d in range(1, N_Z):
        pl.semaphore_signal(
            barrier_sem, inc=1,
            device_id=(my_x, my_y, (my_z + d) % N_Z),
            device_id_type=pl.DeviceIdType.MESH,
        )
    pl.semaphore_wait(barrier_sem, N_Z - 1)

    oc_ref[0] = c_ref[...]
    ouk_ref[0] = uko_ref[...]
    ouv_ref[0] = uvo_ref[...]

    uks = (uk1_ref, uk2_ref, uk3_ref)
    uvs = (uv1_ref, uv2_ref, uv3_ref)
    rdmas = []
    for d in range(1, N_Z):
        peer = (my_z + d) % N_Z
        for t, (src, dst) in enumerate((
            (c_ref, oc_ref),
            (uks[d - 1], ouk_ref),
            (uvs[d - 1], ouv_ref),
        )):
            r = pltpu.make_async_remote_copy(
                src_ref=src,
                dst_ref=dst.at[d],
                send_sem=send_sems.at[d - 1, t],
                recv_sem=recv_sems.at[d - 1, t],
                device_id=(my_x, my_y, peer),
                device_id_type=pl.DeviceIdType.MESH,
            )
            r.start()
            rdmas.append(r)
    for r in rdmas:
        r.wait()


def exchange(c, uk_own, uv_own, uk_s, uv_s):
    n_sem = (N_Z - 1, 3)
    return pl.pallas_call(
        _exch_body,
        out_shape=[
            jax.ShapeDtypeStruct((N_Z, B * S, DC_SH), jnp.bfloat16),
            jax.ShapeDtypeStruct((N_Z, DC_SH, WQ_L), jnp.bfloat16),
            jax.ShapeDtypeStruct((N_Z, DC_SH, WQ_L), jnp.bfloat16),
        ],
        in_specs=[pl.BlockSpec(memory_space=pltpu.VMEM)] * 9,
        out_specs=[pl.BlockSpec(memory_space=pltpu.VMEM)] * 3,
        scratch_shapes=[
            pltpu.SemaphoreType.DMA(n_sem),
            pltpu.SemaphoreType.DMA(n_sem),
        ],
        compiler_params=pltpu.CompilerParams(collective_id=0),
    )(c, uk_own, uv_own, *uk_s, *uv_s)


_SCALE = (Dh + Dr) ** -0.5


_HPP = 2


def _attn_body(q_ref, k_ref, v_ref, qr_ref, kr_ref, o_ref):
    bf = jnp.bfloat16
    q = q_ref[...].astype(bf)
    k = k_ref[...].astype(bf)
    v = v_ref[...].astype(bf)
    qr = qr_ref[...].astype(bf)
    kr = kr_ref[...].astype(bf)
    for i in range(_HPP):
        qh = q[:, i * Dh:(i + 1) * Dh]
        kh = k[:, i * Dh:(i + 1) * Dh]
        vh = v[:, i * Dh:(i + 1) * Dh]
        qrh = qr[:, i * Dr:(i + 1) * Dr]
        s = lax.dot_general(
            qh, kh, (((1,), (1,)), ((), ())),
            preferred_element_type=jnp.float32,
        )
        s += lax.dot_general(
            qrh, kr, (((1,), (1,)), ((), ())),
            preferred_element_type=jnp.float32,
        )
        s *= _SCALE
        m = jnp.max(s, axis=1, keepdims=True)
        p = jnp.exp(s - m)
        p = p / jnp.sum(p, axis=1, keepdims=True)
        o_ref[:, i * Dh:(i + 1) * Dh] = jnp.dot(
            p.astype(bf), vh, preferred_element_type=jnp.float32
        ).astype(bf)


def attention(Q, K, V, Qr, Kr):
    return pl.pallas_call(
        _attn_body,
        grid=(B, H_L // _HPP),
        in_specs=[
            pl.BlockSpec((S, _HPP * Dh), lambda b, h: (b, h)),
            pl.BlockSpec((S, _HPP * Dh), lambda b, h: (b, h)),
            pl.BlockSpec((S, _HPP * Dh), lambda b, h: (b, h)),
            pl.BlockSpec((S, _HPP * Dr), lambda b, h: (b, h)),
            pl.BlockSpec((S, Dr), lambda b, h: (b, 0)),
        ],
        out_specs=pl.BlockSpec((S, _HPP * Dh), lambda b, h: (b, h)),
        out_shape=jax.ShapeDtypeStruct((B * S, H_L * Dh), jnp.bfloat16),
        compiler_params=pltpu.CompilerParams(
            dimension_semantics=("parallel", "parallel")
        ),
    )(Q, K, V, Qr, Kr)


def _agwo_body(o_ref, wo_ref, out_ref, slots, wo_buf,
               send_sems, recv_sems, wo_sem):
    my_x = lax.axis_index("x")
    my_y = lax.axis_index("y")
    my_z = lax.axis_index("z")
    left = (my_z - 1) % N_Z
    right = (my_z + 1) % N_Z

    barrier_sem = pltpu.get_barrier_semaphore()
    for nbr in (left, right):
        pl.semaphore_signal(
            barrier_sem, inc=1,
            device_id=(my_x, my_y, nbr),
            device_id_type=pl.DeviceIdType.MESH,
        )
    pl.semaphore_wait(barrier_sem, 2)

    slots[0] = o_ref[...]

    def wo_dma(h):
        origin = (my_z - h) % N_Z
        return pltpu.make_async_copy(
            wo_ref.at[pl.ds(origin * WQ_L, WQ_L), :], wo_buf, wo_sem
        )

    wo_dma(0).start()
    for h in range(N_Z):
        if h < N_Z - 1:
            rdma = pltpu.make_async_remote_copy(
                src_ref=slots.at[h],
                dst_ref=slots.at[h + 1],
                send_sem=send_sems.at[h],
                recv_sem=recv_sems.at[h],
                device_id=(my_x, my_y, right),
                device_id_type=pl.DeviceIdType.MESH,
            )
            rdma.start()
        wo_dma(h).wait()
        prod = jnp.dot(
            slots[h], wo_buf[...].astype(jnp.bfloat16),
            preferred_element_type=jnp.float32,
        )
        if h == 0:
            out_ref[...] = prod
        else:
            out_ref[...] += prod
        if h < N_Z - 1:
            rdma.wait()
            wo_dma(h + 1).start()


def ag_wo(O_my, Wo):
    return pl.pallas_call(
        _agwo_body,
        out_shape=jax.ShapeDtypeStruct((B * S, D), jnp.float32),
        in_specs=[
            pl.BlockSpec(memory_space=pltpu.VMEM),
            pl.BlockSpec(memory_space=pltpu.MemorySpace.HBM),
        ],
        out_specs=pl.BlockSpec(memory_space=pltpu.VMEM),
        scratch_shapes=[
            pltpu.VMEM((N_Z, B * S, WQ_L), jnp.bfloat16),
            pltpu.VMEM((WQ_L, D), jnp.float32),
            pltpu.SemaphoreType.DMA((N_Z - 1,)),
            pltpu.SemaphoreType.DMA((N_Z - 1,)),
            pltpu.SemaphoreType.DMA,
        ],
        compiler_params=pltpu.CompilerParams(collective_id=1),
    )(O_my, Wo)


def kernel(x, Wdkv, Wuk, Wuv, Wq, Wqr, Wkr, Wo):
    my_z = lax.axis_index("z")
    bf = jnp.bfloat16
    x2 = x.reshape(B * S, D)

    def cols(W, pos, w):
        return lax.dynamic_slice(W, (0, pos * w), (W.shape[0], w)).astype(bf)

    c = mm(x2, Wdkv)

    uk_own = cols(Wuk, my_z, WQ_L)
    uv_own = cols(Wuv, my_z, WQ_L)
    uk_s = [cols(Wuk, (my_z + d) % N_Z, WQ_L) for d in range(1, N_Z)]
    uv_s = [cols(Wuv, (my_z + d) % N_Z, WQ_L) for d in range(1, N_Z)]
    c_all, uk_all, uv_all = exchange(
        c.astype(bf), uk_own, uv_own, uk_s, uv_s
    )

    c_full = c_all.transpose(1, 0, 2).reshape(B * S, N_Z * DC_SH)
    uk_my = uk_all.reshape(N_Z * DC_SH, WQ_L)
    uv_my = uv_all.reshape(N_Z * DC_SH, WQ_L)

    K = mm(c_full, uk_my)
    V = mm(c_full, uv_my)
    Q = mm(x2, Wq, col_off=my_z * WQ_L, n=WQ_L)
    Qr = mm(x2, Wqr, col_off=my_z * WR_L, n=WR_L)
    Kr = mm(x2, Wkr)

    O = attention(Q, K, V, Qr, Kr)

    out = ag_wo(O, Wo)
    return out.reshape(B, S, D)


# device time: 208321 ns/iter; 3.3056x vs baseline; 1.1584x over previous
import jax
import jax.numpy as jnp
from jax import lax
from jax.experimental import pallas as pl
from jax.experimental.pallas import tpu as pltpu

N_Z = 4
N_Y = 4
B, S, H, Dh, Dr = 4, 256, 32, 128, 64
D = 4096
DC_SH = 128
H_L = H // N_Z
WQ_L = H_L * Dh
WR_L = H_L * Dr
H_Y = H_L // N_Y
W_YQ = H_Y * Dh
W_YR = H_Y * Dr


def _mm_body(off_ref, a_ref, b_ref, o_ref):
    @pl.when(pl.program_id(2) == 0)
    def _():
        o_ref[...] = jnp.zeros_like(o_ref)

    a = a_ref[...]
    b = b_ref[...]
    if a.dtype != jnp.bfloat16:
        a = a.astype(jnp.bfloat16)
    if b.dtype != jnp.bfloat16:
        b = b.astype(jnp.bfloat16)
    o_ref[...] += jnp.dot(a, b, preferred_element_type=jnp.float32)


def mm(a, b, col_off=None, n=None, bm=512, bn=512, bk=512):
    M, K = a.shape
    K2, N = b.shape
    assert K == K2
    n = N if n is None else n
    bm, bn, bk = min(bm, M), min(bn, n), min(bk, K)
    if col_off is None:
        col_off = 0
    off_blk = jnp.reshape(col_off // bn, (1,)).astype(jnp.int32)
    grid = (M // bm, n // bn, K // bk)
    return pl.pallas_call(
        _mm_body,
        grid_spec=pltpu.PrefetchScalarGridSpec(
            num_scalar_prefetch=1,
            grid=grid,
            in_specs=[
                pl.BlockSpec((bm, bk), lambda i, j, k, off: (i, k)),
                pl.BlockSpec((bk, bn), lambda i, j, k, off: (k, off[0] + j)),
            ],
            out_specs=pl.BlockSpec((bm, bn), lambda i, j, k, off: (i, j)),
        ),
        out_shape=jax.ShapeDtypeStruct((M, n), jnp.float32),
        compiler_params=pltpu.CompilerParams(
            dimension_semantics=("parallel", "parallel", "arbitrary")
        ),
    )(off_blk, a, b)


def _exch_body(c_ref, uko_ref, uvo_ref,
               uk1_ref, uk2_ref, uk3_ref, uv1_ref, uv2_ref, uv3_ref,
               oc_ref, ouk_ref, ouv_ref, send_sems, recv_sems):
    my_x = lax.axis_index("x")
    my_y = lax.axis_index("y")
    my_z = lax.axis_index("z")

    barrier_sem = pltpu.get_barrier_semaphore()
    for d in range(1, N_Z):
        pl.semaphore_signal(
            barrier_sem, inc=1,
            device_id=(my_x, my_y, (my_z + d) % N_Z),
            device_id_type=pl.DeviceIdType.MESH,
        )
    pl.semaphore_wait(barrier_sem, N_Z - 1)

    oc_ref[0] = c_ref[...]
    ouk_ref[0] = uko_ref[...]
    ouv_ref[0] = uvo_ref[...]

    uks = (uk1_ref, uk2_ref, uk3_ref)
    uvs = (uv1_ref, uv2_ref, uv3_ref)
    rdmas = []
    for d in range(1, N_Z):
        peer = (my_z + d) % N_Z
        for t, (src, dst) in enumerate((
            (c_ref, oc_ref),
            (uks[d - 1], ouk_ref),
            (uvs[d - 1], ouv_ref),
        )):
            r = pltpu.make_async_remote_copy(
                src_ref=src,
                dst_ref=dst.at[d],
                send_sem=send_sems.at[d - 1, t],
                recv_sem=recv_sems.at[d - 1, t],
                device_id=(my_x, my_y, peer),
                device_id_type=pl.DeviceIdType.MESH,
            )
            r.start()
            rdmas.append(r)
    for r in rdmas:
        r.wait()


def exchange(c, uk_own, uv_own, uk_s, uv_s):
    n_sem = (N_Z - 1, 3)
    return pl.pallas_call(
        _exch_body,
        out_shape=[
            jax.ShapeDtypeStruct((N_Z, B * S, DC_SH), jnp.bfloat16),
            jax.ShapeDtypeStruct((N_Z, DC_SH, W_YQ), jnp.bfloat16),
            jax.ShapeDtypeStruct((N_Z, DC_SH, W_YQ), jnp.bfloat16),
        ],
        in_specs=[pl.BlockSpec(memory_space=pltpu.VMEM)] * 9,
        out_specs=[pl.BlockSpec(memory_space=pltpu.VMEM)] * 3,
        scratch_shapes=[
            pltpu.SemaphoreType.DMA(n_sem),
            pltpu.SemaphoreType.DMA(n_sem),
        ],
        compiler_params=pltpu.CompilerParams(collective_id=0),
    )(c, uk_own, uv_own, *uk_s, *uv_s)


_SCALE = (Dh + Dr) ** -0.5


_HPP = 2


def _attn_body(q_ref, k_ref, v_ref, qr_ref, kr_ref, o_ref):
    bf = jnp.bfloat16
    q = q_ref[...].astype(bf)
    k = k_ref[...].astype(bf)
    v = v_ref[...].astype(bf)
    qr = qr_ref[...].astype(bf)
    kr = kr_ref[...].astype(bf)
    for i in range(_HPP):
        qh = q[:, i * Dh:(i + 1) * Dh]
        kh = k[:, i * Dh:(i + 1) * Dh]
        vh = v[:, i * Dh:(i + 1) * Dh]
        qrh = qr[:, i * Dr:(i + 1) * Dr]
        s = lax.dot_general(
            qh, kh, (((1,), (1,)), ((), ())),
            preferred_element_type=jnp.float32,
        )
        s += lax.dot_general(
            qrh, kr, (((1,), (1,)), ((), ())),
            preferred_element_type=jnp.float32,
        )
        s *= _SCALE
        m = jnp.max(s, axis=1, keepdims=True)
        p = jnp.exp(s - m)
        p = p / jnp.sum(p, axis=1, keepdims=True)
        o_ref[:, i * Dh:(i + 1) * Dh] = jnp.dot(
            p.astype(bf), vh, preferred_element_type=jnp.float32
        ).astype(bf)


def attention(Q, K, V, Qr, Kr):
    return pl.pallas_call(
        _attn_body,
        grid=(B, H_Y // _HPP),
        in_specs=[
            pl.BlockSpec((S, _HPP * Dh), lambda b, h: (b, h)),
            pl.BlockSpec((S, _HPP * Dh), lambda b, h: (b, h)),
            pl.BlockSpec((S, _HPP * Dh), lambda b, h: (b, h)),
            pl.BlockSpec((S, _HPP * Dr), lambda b, h: (b, h)),
            pl.BlockSpec((S, Dr), lambda b, h: (b, 0)),
        ],
        out_specs=pl.BlockSpec((S, _HPP * Dh), lambda b, h: (b, h)),
        out_shape=jax.ShapeDtypeStruct((B * S, H_Y * Dh), jnp.bfloat16),
        compiler_params=pltpu.CompilerParams(
            dimension_semantics=("parallel", "parallel")
        ),
    )(Q, K, V, Qr, Kr)


def _yag_body(o_ref, slots_ref, send_sems, recv_sems):
    my_x = lax.axis_index("x")
    my_y = lax.axis_index("y")
    my_z = lax.axis_index("z")
    left = (my_y - 1) % N_Y
    right = (my_y + 1) % N_Y

    barrier_sem = pltpu.get_barrier_semaphore()
    for nbr in (left, right):
        pl.semaphore_signal(
            barrier_sem, inc=1,
            device_id=(my_x, nbr, my_z),
            device_id_type=pl.DeviceIdType.MESH,
        )
    pl.semaphore_wait(barrier_sem, 2)

    slots_ref[0] = o_ref[...]
    for h in range(N_Y - 1):
        rdma = pltpu.make_async_remote_copy(
            src_ref=slots_ref.at[h],
            dst_ref=slots_ref.at[h + 1],
            send_sem=send_sems.at[h],
            recv_sem=recv_sems.at[h],
            device_id=(my_x, right, my_z),
            device_id_type=pl.DeviceIdType.MESH,
        )
        rdma.start()
        rdma.wait()


def y_allgather(O_my):
    return pl.pallas_call(
        _yag_body,
        out_shape=jax.ShapeDtypeStruct((N_Y, B * S, W_YQ), jnp.bfloat16),
        in_specs=[pl.BlockSpec(memory_space=pltpu.VMEM)],
        out_specs=pl.BlockSpec(memory_space=pltpu.VMEM),
        scratch_shapes=[
            pltpu.SemaphoreType.DMA((N_Y - 1,)),
            pltpu.SemaphoreType.DMA((N_Y - 1,)),
        ],
        compiler_params=pltpu.CompilerParams(collective_id=2),
    )(O_my)


def _agwo_body(o_ref, wo_ref, out_ref, slots, wo_buf,
               send_sems, recv_sems, wo_sem):
    my_x = lax.axis_index("x")
    my_y = lax.axis_index("y")
    my_z = lax.axis_index("z")
    left = (my_z - 1) % N_Z
    right = (my_z + 1) % N_Z

    barrier_sem = pltpu.get_barrier_semaphore()
    for nbr in (left, right):
        pl.semaphore_signal(
            barrier_sem, inc=1,
            device_id=(my_x, my_y, nbr),
            device_id_type=pl.DeviceIdType.MESH,
        )
    pl.semaphore_wait(barrier_sem, 2)

    slots[0] = o_ref[...]

    def wo_dmas(h):
        origin = (my_z - h) % N_Z
        copies = []
        for j in range(N_Y):
            y_org = (my_y - j) % N_Y
            copies.append(pltpu.make_async_copy(
                wo_ref.at[pl.ds(origin * WQ_L + y_org * W_YQ, W_YQ), :],
                wo_buf.at[pl.ds(j * W_YQ, W_YQ), :],
                wo_sem.at[j],
            ))
        return copies

    for cp in wo_dmas(0):
        cp.start()
    for h in range(N_Z):
        if h < N_Z - 1:
            rdma = pltpu.make_async_remote_copy(
                src_ref=slots.at[h],
                dst_ref=slots.at[h + 1],
                send_sem=send_sems.at[h],
                recv_sem=recv_sems.at[h],
                device_id=(my_x, my_y, right),
                device_id_type=pl.DeviceIdType.MESH,
            )
            rdma.start()
        for cp in wo_dmas(h):
            cp.wait()
        prod = jnp.dot(
            slots[h], wo_buf[...].astype(jnp.bfloat16),
            preferred_element_type=jnp.float32,
        )
        if h == 0:
            out_ref[...] = prod
        else:
            out_ref[...] += prod
        if h < N_Z - 1:
            rdma.wait()
            for cp in wo_dmas(h + 1):
                cp.start()


def ag_wo(O_my, Wo):
    return pl.pallas_call(
        _agwo_body,
        out_shape=jax.ShapeDtypeStruct((B * S, D), jnp.float32),
        in_specs=[
            pl.BlockSpec(memory_space=pltpu.VMEM),
            pl.BlockSpec(memory_space=pltpu.MemorySpace.HBM),
        ],
        out_specs=pl.BlockSpec(memory_space=pltpu.VMEM),
        scratch_shapes=[
            pltpu.VMEM((N_Z, B * S, WQ_L), jnp.bfloat16),
            pltpu.VMEM((WQ_L, D), jnp.float32),
            pltpu.SemaphoreType.DMA((N_Z - 1,)),
            pltpu.SemaphoreType.DMA((N_Z - 1,)),
            pltpu.SemaphoreType.DMA((N_Y,)),
        ],
        compiler_params=pltpu.CompilerParams(collective_id=1),
    )(O_my, Wo)


def kernel(x, Wdkv, Wuk, Wuv, Wq, Wqr, Wkr, Wo):
    my_y = lax.axis_index("y")
    my_z = lax.axis_index("z")
    bf = jnp.bfloat16
    x2 = x.reshape(B * S, D)

    def head_cols(W, z_pos, w_z, w_y):
        return lax.dynamic_slice(
            W, (0, z_pos * w_z + my_y * w_y), (W.shape[0], w_y)
        ).astype(bf)

    c = mm(x2, Wdkv)

    uk_own = head_cols(Wuk, my_z, WQ_L, W_YQ)
    uv_own = head_cols(Wuv, my_z, WQ_L, W_YQ)
    uk_s = [head_cols(Wuk, (my_z + d) % N_Z, WQ_L, W_YQ)
            for d in range(1, N_Z)]
    uv_s = [head_cols(Wuv, (my_z + d) % N_Z, WQ_L, W_YQ)
            for d in range(1, N_Z)]
    c_all, uk_all, uv_all = exchange(
        c.astype(bf), uk_own, uv_own, uk_s, uv_s
    )

    c_full = c_all.transpose(1, 0, 2).reshape(B * S, N_Z * DC_SH)
    uk_my = uk_all.reshape(N_Z * DC_SH, W_YQ)
    uv_my = uv_all.reshape(N_Z * DC_SH, W_YQ)

    K = mm(c_full, uk_my)
    V = mm(c_full, uv_my)
    Q = mm(x2, Wq, col_off=my_z * WQ_L + my_y * W_YQ, n=W_YQ)
    Qr = mm(x2, Wqr, col_off=my_z * WR_L + my_y * W_YR, n=W_YR)
    Kr = mm(x2, Wkr)

    O = attention(Q, K, V, Qr, Kr)

    O_all = y_allgather(O)
    O_cat = O_all.transpose(1, 0, 2).reshape(B * S, WQ_L)

    out = ag_wo(O_cat, Wo)
    return out.reshape(B, S, D)
